# baseline (speedup 1.0000x reference)
import functools

import jax
import jax.numpy as jnp
from jax.experimental import pallas as pl
from jax.experimental.pallas import tpu as pltpu


def _router_kernel(x_ref, w_ref, b_ref, out_ref, acc_ref, *, num_experts):
    k = pl.program_id(0)

    @pl.when(k == 0)
    def _init():
        acc_ref[...] = jnp.zeros_like(acc_ref)

    acc_ref[...] += jax.lax.dot_general(
        x_ref[...], w_ref[...],
        dimension_numbers=(((1,), (1,)), ((), ())),
        preferred_element_type=jnp.float32,
    )

    @pl.when(k == pl.num_programs(0) - 1)
    def _finish():
        logits = acc_ref[...] + b_ref[...]
        mx = jnp.max(logits, axis=1, keepdims=True)
        ids = jax.lax.broadcasted_iota(jnp.int32, logits.shape, 1)
        idx = jnp.min(jnp.where(logits == mx, ids, num_experts), axis=1)
        out_ref[...] = idx.astype(jnp.int32)[:, None]


def _pick_bk(k_total):
    for bk in (3072, 2048, 1024, 512, 256, 128):
        if k_total % bk == 0:
            return bk
    return k_total


@jax.jit
def kernel(x, W, b):
    batch = x.shape[0]
    num_experts = W.shape[0]
    xf = x.reshape(batch, -1)
    k_total = xf.shape[1]
    bk = _pick_bk(k_total)
    steps = k_total // bk

    out = pl.pallas_call(
        functools.partial(_router_kernel, num_experts=num_experts),
        grid=(steps,),
        in_specs=[
            pl.BlockSpec((batch, bk), lambda k: (0, k)),
            pl.BlockSpec((num_experts, bk), lambda k: (0, k)),
            pl.BlockSpec((1, num_experts), lambda k: (0, 0)),
        ],
        out_specs=pl.BlockSpec((batch, 1), lambda k: (0, 0)),
        out_shape=jax.ShapeDtypeStruct((batch, 1), jnp.int32),
        scratch_shapes=[pltpu.VMEM((batch, num_experts), jnp.float32)],
        compiler_params=pltpu.CompilerParams(
            dimension_semantics=("arbitrary",),
        ),
    )(xf, W, b.reshape(1, num_experts))
    return out.reshape(batch)
